# baseline (speedup 1.0000x reference)
import functools

import jax
import jax.numpy as jnp
from jax import lax
from jax.experimental import pallas as pl
from jax.experimental.pallas import tpu as pltpu
from jax.experimental.pallas import tpu_sc as plsc

BATCH = 4096
NUM_WORDS = 50
EMB = 32
LAT = 64
HALF = 16

NC = 2
NS = 16
NW = NC * NS
B_PER_W = BATCH // NW
CHUNK = 16
NCHUNK = B_PER_W // CHUNK
ROWS = CHUNK * NUM_WORDS

_mesh = plsc.VectorSubcoreMesh(core_axis_name="c", subcore_axis_name="s")


def _pool_row(buf, base):
    acc0 = [None] * 4
    acc1 = [None] * 4
    for k in range(NUM_WORDS):
        v0 = buf[base + k, pl.ds(0, HALF)]
        v1 = buf[base + k, pl.ds(HALF, HALF)]
        g = k % 4
        acc0[g] = v0 if acc0[g] is None else acc0[g] + v0
        acc1[g] = v1 if acc1[g] is None else acc1[g] + v1
    s0 = (acc0[0] + acc0[1]) + (acc0[2] + acc0[3])
    s1 = (acc1[0] + acc1[1]) + (acc1[2] + acc1[3])
    scale = jnp.float32(1.0 / NUM_WORDS)
    return s0 * scale, s1 * scale


@functools.partial(
    pl.kernel,
    mesh=_mesh,
    compiler_params=pltpu.CompilerParams(use_tc_tiling_on_sc=False),
    out_type=jax.ShapeDtypeStruct((BATCH, EMB), jnp.float32),
    scratch_types=[
        pltpu.VMEM((B_PER_W * NUM_WORDS,), jnp.int32),
        pltpu.VMEM((ROWS, EMB), jnp.float32),
        pltpu.VMEM((ROWS, EMB), jnp.float32),
        pltpu.VMEM((B_PER_W, EMB), jnp.float32),
        pltpu.SemaphoreType.DMA,
        pltpu.SemaphoreType.DMA,
    ],
)
def _sc_pool(w_hbm, table_hbm, out_hbm, idx_v, buf0, buf1, out_v, sem0, sem1):
    wid = lax.axis_index("s") * NC + lax.axis_index("c")
    ibase = wid * (B_PER_W * NUM_WORDS)
    obase = wid * B_PER_W

    pltpu.sync_copy(w_hbm.at[pl.ds(ibase, B_PER_W * NUM_WORDS)], idx_v)

    bufs = (buf0, buf1)
    sems = (sem0, sem1)
    handles = [None, None]

    def start(c):
        idx_sl = idx_v.at[pl.ds(c * ROWS, ROWS)]
        handles[c % 2] = pltpu.async_copy(
            table_hbm.at[idx_sl], bufs[c % 2], sems[c % 2])

    def reduce_chunk(c):
        buf = bufs[c % 2]

        def row_body(r, carry):
            s0, s1 = _pool_row(buf, r * NUM_WORDS)
            orow = c * CHUNK + r
            out_v[orow, pl.ds(0, HALF)] = s0
            out_v[orow, pl.ds(HALF, HALF)] = s1
            return carry

        lax.fori_loop(0, CHUNK, row_body, 0)

    start(0)
    for c in range(1, NCHUNK):
        start(c)
        handles[(c - 1) % 2].wait()
        reduce_chunk(c - 1)
    handles[(NCHUNK - 1) % 2].wait()
    reduce_chunk(NCHUNK - 1)

    pltpu.sync_copy(out_v, out_hbm.at[pl.ds(obase, B_PER_W)])


def _heads_body(p_ref, wmu_ref, bmu_ref, wsig_ref, bsig_ref, mean_ref, logstd_ref):
    p = p_ref[...]
    mean_ref[...] = (
        jnp.dot(p, wmu_ref[...], preferred_element_type=jnp.float32)
        + bmu_ref[...]
    )
    logstd_ref[...] = (
        jnp.dot(p, wsig_ref[...], preferred_element_type=jnp.float32)
        + bsig_ref[...]
    )


_heads = pl.pallas_call(
    _heads_body,
    out_shape=(
        jax.ShapeDtypeStruct((BATCH, LAT), jnp.float32),
        jax.ShapeDtypeStruct((BATCH, LAT), jnp.float32),
    ),
)


def kernel(w, table, W_mu, b_mu, W_sig, b_sig):
    w_flat = w.reshape(-1).astype(jnp.int32)
    pooled = _sc_pool(w_flat, table)
    mean, logstd = _heads(
        pooled, W_mu, b_mu.reshape(1, LAT), W_sig, b_sig.reshape(1, LAT))
    return (mean, logstd)
